# baseline (speedup 1.0000x reference)
import functools

import jax
import jax.numpy as jnp
from jax.experimental import pallas as pl
from jax.experimental.pallas import tpu as pltpu

N = 10000
E = 160000
D_IN = 128
D_HID = 128
D_OUT = 16
NEG_SLOPE = 0.2
BLK = 1000


def _dense1_kernel(x_ref, mW1_ref, mb1_ref, mW2_ref, mb2_ref, mW3_ref, mb3_ref,
                   pars_ref, W0_ref, as_ref, ad_ref,
                   logits_ref, ql_ref, h0_ref, s_ref, t_ref):
    x = x_ref[...]
    p = jnp.maximum(x @ mW1_ref[...] + mb1_ref[...], 0.0)
    p = jnp.maximum(p @ mW2_ref[...] + mb2_ref[...], 0.0)
    lg = p @ mW3_ref[...] + mb3_ref[...]
    logits_ref[...] = lg
    pars = jnp.maximum(2.0 * pars_ref[...], 0.0)
    ql_ref[...] = lg @ pars
    h = x @ W0_ref[...]
    h0_ref[...] = h
    s_ref[...] = jnp.sum(h * as_ref[...], axis=1, keepdims=True)
    t_ref[...] = jnp.sum(h * ad_ref[...], axis=1, keepdims=True)


def _dense2_kernel(o_ref, b0_ref, W1_ref, as_ref, ad_ref,
                   h1_ref, s_ref, t_ref):
    h = jnp.maximum(o_ref[...] + b0_ref[...], 0.0)
    h1 = h @ W1_ref[...]
    h1_ref[...] = h1
    s_ref[...] = jnp.sum(h1 * as_ref[...], axis=1, keepdims=True)
    t_ref[...] = jnp.sum(h1 * ad_ref[...], axis=1, keepdims=True)


def _edges0_kernel(pk_ref, s_ref, t_ref, logits_ref, ql_ref, h_ref,
                   out_ref, mean_ref, scale_ref, den_ref):
    den_ref[...] = jnp.zeros_like(den_ref)
    out_ref[...] = jnp.zeros_like(out_ref)
    mg = jnp.max(s_ref[...]) + jnp.max(t_ref[...])
    mg = jnp.maximum(mg, NEG_SLOPE * mg)

    def body1(i, carry):
        sew, sew2 = carry
        p = pk_ref[i]
        si = p // 32768
        di = p - si * 32768
        e = s_ref[pl.ds(si, 1), :] + t_ref[pl.ds(di, 1), :]
        e = jnp.where(e > 0, e, NEG_SLOPE * e)
        ex = jnp.exp(e - mg)
        den_ref[pl.ds(di, 1), :] = den_ref[pl.ds(di, 1), :] + ex
        ew = jnp.sum(logits_ref[pl.ds(si, 1), :] * ql_ref[pl.ds(di, 1), :],
                     axis=1, keepdims=True)
        return (sew + ew, sew2 + ew * ew)

    z = jnp.zeros((1, 1), jnp.float32)
    sew, sew2 = jax.lax.fori_loop(0, E, body1, (z, z))
    mean = sew / E
    var = (sew2 - sew * sew / E) / (E - 1)
    scale = jnp.sqrt(1e-4 / var)
    mean_ref[...] = mean
    scale_ref[...] = scale

    def body2(i, _):
        p = pk_ref[i]
        si = p // 32768
        di = p - si * 32768
        e = s_ref[pl.ds(si, 1), :] + t_ref[pl.ds(di, 1), :]
        e = jnp.where(e > 0, e, NEG_SLOPE * e)
        ex = jnp.exp(e - mg)
        ew = jnp.sum(logits_ref[pl.ds(si, 1), :] * ql_ref[pl.ds(di, 1), :],
                     axis=1, keepdims=True)
        ewn = (ew - mean) * scale + 1.0
        alpha = ex / (den_ref[pl.ds(di, 1), :] + 1e-16) * ewn
        out_ref[pl.ds(di, 1), :] = (out_ref[pl.ds(di, 1), :]
                                    + alpha * h_ref[pl.ds(si, 1), :])
        return 0

    jax.lax.fori_loop(0, E, body2, 0)


def _edges1_kernel(pk_ref, s_ref, t_ref, logits_ref, ql_ref, h_ref,
                   mean_ref, scale_ref, b1_ref, out_ref, den_ref):
    den_ref[...] = jnp.zeros_like(den_ref)
    out_ref[...] = jnp.zeros_like(out_ref)
    mg = jnp.max(s_ref[...]) + jnp.max(t_ref[...])
    mg = jnp.maximum(mg, NEG_SLOPE * mg)
    mean = mean_ref[...]
    scale = scale_ref[...]

    def body1(i, _):
        p = pk_ref[i]
        si = p // 32768
        di = p - si * 32768
        e = s_ref[pl.ds(si, 1), :] + t_ref[pl.ds(di, 1), :]
        e = jnp.where(e > 0, e, NEG_SLOPE * e)
        ex = jnp.exp(e - mg)
        den_ref[pl.ds(di, 1), :] = den_ref[pl.ds(di, 1), :] + ex
        return 0

    jax.lax.fori_loop(0, E, body1, 0)

    def body2(i, _):
        p = pk_ref[i]
        si = p // 32768
        di = p - si * 32768
        e = s_ref[pl.ds(si, 1), :] + t_ref[pl.ds(di, 1), :]
        e = jnp.where(e > 0, e, NEG_SLOPE * e)
        ex = jnp.exp(e - mg)
        ew = jnp.sum(logits_ref[pl.ds(si, 1), :] * ql_ref[pl.ds(di, 1), :],
                     axis=1, keepdims=True)
        ewn = (ew - mean) * scale + 1.0
        alpha = ex / (den_ref[pl.ds(di, 1), :] + 1e-16) * ewn
        out_ref[pl.ds(di, 1), :] = (out_ref[pl.ds(di, 1), :]
                                    + alpha * h_ref[pl.ds(si, 1), :])
        return 0

    jax.lax.fori_loop(0, E, body2, 0)
    out_ref[...] = out_ref[...] + b1_ref[...]


@jax.jit
def kernel(x, edge_index, W0, a_src0, a_dst0, b0, W1, a_src1, a_dst1, b1,
           mW1, mb1, mW2, mb2, mW3, mb3, parsing0):
    packed = edge_index[0] * 32768 + edge_index[1]
    f32 = jnp.float32
    grid = N // BLK

    rep = lambda i: (0, 0)
    blk = lambda i: (i, 0)

    logits, ql, h0, s0, t0 = pl.pallas_call(
        _dense1_kernel,
        grid=(grid,),
        in_specs=[
            pl.BlockSpec((BLK, D_IN), blk),
            pl.BlockSpec((D_IN, 512), rep),
            pl.BlockSpec((1, 512), rep),
            pl.BlockSpec((512, 64), rep),
            pl.BlockSpec((1, 64), rep),
            pl.BlockSpec((64, D_OUT), rep),
            pl.BlockSpec((1, D_OUT), rep),
            pl.BlockSpec((D_OUT, D_OUT), rep),
            pl.BlockSpec((D_IN, D_HID), rep),
            pl.BlockSpec((1, D_HID), rep),
            pl.BlockSpec((1, D_HID), rep),
        ],
        out_specs=[
            pl.BlockSpec((BLK, D_OUT), blk),
            pl.BlockSpec((BLK, D_OUT), blk),
            pl.BlockSpec((BLK, D_HID), blk),
            pl.BlockSpec((BLK, 1), blk),
            pl.BlockSpec((BLK, 1), blk),
        ],
        out_shape=[
            jax.ShapeDtypeStruct((N, D_OUT), f32),
            jax.ShapeDtypeStruct((N, D_OUT), f32),
            jax.ShapeDtypeStruct((N, D_HID), f32),
            jax.ShapeDtypeStruct((N, 1), f32),
            jax.ShapeDtypeStruct((N, 1), f32),
        ],
    )(x, mW1, mb1.reshape(1, -1), mW2, mb2.reshape(1, -1), mW3,
      mb3.reshape(1, -1), parsing0, W0, a_src0.reshape(1, -1),
      a_dst0.reshape(1, -1))

    rep2 = lambda i, p: (0, 0)
    out0, mean, scale = pl.pallas_call(
        _edges0_kernel,
        grid_spec=pltpu.PrefetchScalarGridSpec(
            num_scalar_prefetch=1,
            grid=(1,),
            in_specs=[
                pl.BlockSpec((N, 1), rep2),
                pl.BlockSpec((N, 1), rep2),
                pl.BlockSpec((N, D_OUT), rep2),
                pl.BlockSpec((N, D_OUT), rep2),
                pl.BlockSpec((N, D_HID), rep2),
            ],
            out_specs=[
                pl.BlockSpec((N, D_HID), rep2),
                pl.BlockSpec((1, 1), rep2),
                pl.BlockSpec((1, 1), rep2),
            ],
            scratch_shapes=[pltpu.VMEM((N, 1), f32)],
        ),
        out_shape=[
            jax.ShapeDtypeStruct((N, D_HID), f32),
            jax.ShapeDtypeStruct((1, 1), f32),
            jax.ShapeDtypeStruct((1, 1), f32),
        ],
    )(packed, s0, t0, logits, ql, h0)

    h1, s1, t1 = pl.pallas_call(
        _dense2_kernel,
        grid=(grid,),
        in_specs=[
            pl.BlockSpec((BLK, D_HID), blk),
            pl.BlockSpec((1, D_HID), rep),
            pl.BlockSpec((D_HID, D_OUT), rep),
            pl.BlockSpec((1, D_OUT), rep),
            pl.BlockSpec((1, D_OUT), rep),
        ],
        out_specs=[
            pl.BlockSpec((BLK, D_OUT), blk),
            pl.BlockSpec((BLK, 1), blk),
            pl.BlockSpec((BLK, 1), blk),
        ],
        out_shape=[
            jax.ShapeDtypeStruct((N, D_OUT), f32),
            jax.ShapeDtypeStruct((N, 1), f32),
            jax.ShapeDtypeStruct((N, 1), f32),
        ],
    )(out0, b0.reshape(1, -1), W1, a_src1.reshape(1, -1),
      a_dst1.reshape(1, -1))

    out = pl.pallas_call(
        _edges1_kernel,
        grid_spec=pltpu.PrefetchScalarGridSpec(
            num_scalar_prefetch=1,
            grid=(1,),
            in_specs=[
                pl.BlockSpec((N, 1), rep2),
                pl.BlockSpec((N, 1), rep2),
                pl.BlockSpec((N, D_OUT), rep2),
                pl.BlockSpec((N, D_OUT), rep2),
                pl.BlockSpec((N, D_OUT), rep2),
                pl.BlockSpec((1, 1), rep2),
                pl.BlockSpec((1, 1), rep2),
                pl.BlockSpec((1, D_OUT), rep2),
            ],
            out_specs=pl.BlockSpec((N, D_OUT), rep2),
            scratch_shapes=[pltpu.VMEM((N, 1), f32)],
        ),
        out_shape=jax.ShapeDtypeStruct((N, D_OUT), f32),
    )(packed, s1, t1, logits, ql, h1, mean, scale, b1.reshape(1, -1))

    return out
